# Initial kernel scaffold; baseline (speedup 1.0000x reference)
import jax
import jax.numpy as jnp
from jax.experimental import pallas as pl


def kernel(h, edge_index, W_feat, b_feat, W_pool, b_pool):
    raise NotImplementedError("write your pallas kernel here")


# speedup vs baseline: 3.2370x; 3.2370x over previous
import jax
import jax.numpy as jnp
from jax import lax
from jax.experimental import pallas as pl
from jax.experimental.pallas import tpu as pltpu
from jax.experimental.pallas import tpu_sc as plsc

N = 10000
E = 320000
D = 128
OUT = 128
ASSIGN = 500
APAD = 512
NBLK = APAD // 128
WCOLS = OUT + APAD

NC = 2
NS = 16
NW = NC * NS
NP = 10240
SR = NP // NS
CH = 80

_MESH = plsc.VectorSubcoreMesh(core_axis_name="c", subcore_axis_name="s")


def _seg_h_body(tab, gidx, sidx, zeros, ones, out, gbuf, sbuf, rows, acc, sem):
    c = lax.axis_index("c")
    s = lax.axis_index("s")
    eps = E // NS
    pltpu.sync_copy(zeros, acc.at[pl.ds(s * SR, SR)])
    plsc.subcore_barrier()

    @pl.when(c == 0)
    def _agg():
        def chunk(k, carry):
            base = pl.multiple_of(s * eps + k * CH, 8)
            pltpu.sync_copy(gidx.at[pl.ds(base, CH)], gbuf)
            pltpu.sync_copy(sidx.at[pl.ds(base, CH)], sbuf)
            pltpu.async_copy(tab.at[gbuf], rows, sem).wait()
            pltpu.sync_copy(rows, acc.at[sbuf], add=True)
            return carry

        lax.fori_loop(0, eps // CH, chunk, 0)

    @pl.when(c == 1)
    def _deg():
        pltpu.sync_copy(ones, rows)

        def chunk(k, carry):
            base = pl.multiple_of(s * eps + k * CH, 8)
            pltpu.sync_copy(sidx.at[pl.ds(base, CH)], sbuf)
            pltpu.sync_copy(rows, acc.at[sbuf], add=True)
            return carry

        lax.fori_loop(0, eps // CH, chunk, 0)

    plsc.subcore_barrier()
    pltpu.sync_copy(acc.at[pl.ds(s * SR, SR)], out.at[c, pl.ds(s * SR, SR)])


_seg_h = pl.kernel(
    _seg_h_body,
    out_type=jax.ShapeDtypeStruct((NC, NP, D), jnp.float32),
    mesh=_MESH,
    scratch_types=[
        pltpu.VMEM((CH,), jnp.int32),
        pltpu.VMEM((CH,), jnp.int32),
        pltpu.VMEM((CH, D), jnp.float32),
        pltpu.VMEM_SHARED((NP, D), jnp.float32),
        pltpu.SemaphoreType.DMA,
    ],
)


def _seg_a_body(tab, gidx, sidx, zeros, out, gbuf, dbuf, sbuf, rows, acc, sem):
    c = lax.axis_index("c")
    s = lax.axis_index("s")
    eps = E // NS
    for bi in range(NBLK // NC):
        b = c * (NBLK // NC) + bi
        off = b * N
        pltpu.sync_copy(zeros, acc.at[pl.ds(s * SR, SR)])
        plsc.subcore_barrier()

        def chunk(k, carry):
            base = pl.multiple_of(s * eps + k * CH, 8)
            pltpu.sync_copy(gidx.at[pl.ds(base, CH)], dbuf)
            pltpu.sync_copy(sidx.at[pl.ds(base, CH)], sbuf)
            for i in range(CH // 16):
                gbuf[pl.ds(i * 16, 16)] = dbuf[pl.ds(i * 16, 16)] + off
            pltpu.async_copy(tab.at[gbuf], rows, sem).wait()
            pltpu.sync_copy(rows, acc.at[sbuf], add=True)
            return carry

        lax.fori_loop(0, eps // CH, chunk, 0)
        plsc.subcore_barrier()
        pltpu.sync_copy(acc.at[pl.ds(s * SR, SR)], out.at[b, pl.ds(s * SR, SR)])


_seg_a = pl.kernel(
    _seg_a_body,
    out_type=jax.ShapeDtypeStruct((NBLK, NP, 128), jnp.float32),
    mesh=_MESH,
    scratch_types=[
        pltpu.VMEM((CH,), jnp.int32),
        pltpu.VMEM((CH,), jnp.int32),
        pltpu.VMEM((CH,), jnp.int32),
        pltpu.VMEM((CH, 128), jnp.float32),
        pltpu.VMEM_SHARED((NP, 128), jnp.float32),
        pltpu.SemaphoreType.DMA,
    ],
)


TRA = 1000


def _dense_body(h_ref, agg_ref, w_ref, b_ref, feat_ref, asn_ref):
    deg = jnp.maximum(agg_ref[1][:, :1], 1.0)
    cmean = agg_ref[0] / deg
    y = jnp.dot(h_ref[...], w_ref[:D, :], preferred_element_type=jnp.float32)
    y = y + jnp.dot(cmean, w_ref[D:, :], preferred_element_type=jnp.float32)
    y = jnp.maximum(y + b_ref[...], 0.0)
    feat_ref[...] = y[:, :OUT]
    p = y[:, OUT:]
    valid = lax.broadcasted_iota(jnp.int32, p.shape, 1) < ASSIGN
    p = jnp.where(valid, p, -1e30)
    m = jnp.max(p, axis=1, keepdims=True)
    e = jnp.exp(p - m)
    a = e / jnp.sum(e, axis=1, keepdims=True)
    for bb in range(NBLK):
        asn_ref[bb] = a[:, bb * 128:(bb + 1) * 128]


_dense = pl.pallas_call(
    _dense_body,
    grid=(N // TRA,),
    in_specs=[
        pl.BlockSpec((TRA, D), lambda i: (i, 0)),
        pl.BlockSpec((NC, TRA, D), lambda i: (0, i, 0)),
        pl.BlockSpec((2 * D, WCOLS), lambda i: (0, 0)),
        pl.BlockSpec((1, WCOLS), lambda i: (0, 0)),
    ],
    out_specs=[
        pl.BlockSpec((TRA, OUT), lambda i: (i, 0)),
        pl.BlockSpec((NBLK, TRA, 128), lambda i: (0, i, 0)),
    ],
    out_shape=[
        jax.ShapeDtypeStruct((N, OUT), jnp.float32),
        jax.ShapeDtypeStruct((NBLK, N, 128), jnp.float32),
    ],
)


TRB = 1000


def _contract_body(asn_ref, feat_ref, as_ref, hn_ref, adj_ref):
    i = pl.program_id(0)

    @pl.when(i == 0)
    def _init():
        hn_ref[...] = jnp.zeros_like(hn_ref)
        adj_ref[...] = jnp.zeros_like(adj_ref)

    f = feat_ref[...]
    dn = (((0,), (0,)), ((), ()))
    for bi in range(NBLK):
        a = asn_ref[bi]
        hn_ref[bi] += lax.dot_general(a, f, dn, preferred_element_type=jnp.float32)
        for bj in range(NBLK):
            adj_ref[bi, bj] += lax.dot_general(
                a, as_ref[bj], dn, preferred_element_type=jnp.float32)


_contract = pl.pallas_call(
    _contract_body,
    grid=(N // TRB,),
    in_specs=[
        pl.BlockSpec((NBLK, TRB, 128), lambda i: (0, i, 0)),
        pl.BlockSpec((TRB, OUT), lambda i: (i, 0)),
        pl.BlockSpec((NBLK, TRB, 128), lambda i: (0, i, 0)),
    ],
    out_specs=[
        pl.BlockSpec((NBLK, 128, OUT), lambda i: (0, 0, 0)),
        pl.BlockSpec((NBLK, NBLK, 128, 128), lambda i: (0, 0, 0, 0)),
    ],
    out_shape=[
        jax.ShapeDtypeStruct((NBLK, 128, OUT), jnp.float32),
        jax.ShapeDtypeStruct((NBLK, NBLK, 128, 128), jnp.float32),
    ],
)


def kernel(h, edge_index, W_feat, b_feat, W_pool, b_pool):
    src = edge_index[0]
    dst = edge_index[1]

    zeros_blk = jnp.zeros((SR, 128), jnp.float32)
    ones_blk = jnp.ones((CH, 128), jnp.float32)
    agg2 = _seg_h(h, src, dst, zeros_blk, ones_blk)

    w_cat = jnp.concatenate(
        [W_feat, W_pool, jnp.zeros((2 * D, APAD - ASSIGN), jnp.float32)], axis=1)
    b_cat = jnp.concatenate(
        [b_feat, b_pool, jnp.zeros((APAD - ASSIGN,), jnp.float32)])[None, :]
    feat, asn = _dense(h, agg2, w_cat, b_cat)

    a_s = _seg_a(asn.reshape(NBLK * N, 128), dst, src, zeros_blk)

    hn_pad, adj_pad = _contract(asn, feat, a_s)
    h_new = hn_pad.reshape(APAD, OUT)[:ASSIGN]
    adj_new = adj_pad.transpose(0, 2, 1, 3).reshape(APAD, APAD)[:ASSIGN, :ASSIGN]
    return (adj_new, h_new)
